# Initial kernel scaffold; baseline (speedup 1.0000x reference)
import jax
import jax.numpy as jnp
from jax.experimental import pallas as pl


def kernel(x_author, x_paper, ei_writes, ei_cites, ei_rev, Wl_0_writes, bl_0_writes, Wr_0_writes, Wl_0_cites, bl_0_cites, Wr_0_cites, Wl_0_rev, bl_0_rev, Wr_0_rev, Wl_1_writes, bl_1_writes, Wr_1_writes, Wl_1_cites, bl_1_cites, Wr_1_cites, Wl_1_rev, bl_1_rev, Wr_1_rev):
    raise NotImplementedError("write your pallas kernel here")


# speedup vs baseline: 3.9389x; 3.9389x over previous
import functools

import jax
import jax.numpy as jnp
from jax import lax
from jax.experimental import pallas as pl
from jax.experimental.pallas import tpu as pltpu
from jax.experimental.pallas import tpu_sc as plsc

N = 10000
D = 256
H = 128
E = 160000
EPAD = 163840
EROWS = EPAD // 128
ROWS_PER_TILE = EROWS // 16
KJ = 8
NGROUP = ROWS_PER_TILE // KJ
NPADROW = 16
NACC = 10112
ZROWS = NACC // 16
RB = 1000
ALPHA = 0.9999950000374997
NEG = 0.01

_MESH = plsc.VectorSubcoreMesh(core_axis_name="c", subcore_axis_name="s",
                               num_cores=2, num_subcores=16)


def _dense_body(a_ref, p_ref, wa_ref, wp_ref,
                yw0, yw1, za, yc0, yc1, yr0, yr1, zp):
    A = jnp.dot(a_ref[...], wa_ref[...], preferred_element_type=jnp.float32)
    P = jnp.dot(p_ref[...], wp_ref[...], preferred_element_type=jnp.float32)
    yw0[...] = A[:, 0:H]
    yw1[...] = A[:, H:2 * H]
    za[...] = A[:, 2 * H:2 * H + D]
    yc0[...] = P[:, 0:H]
    yc1[...] = P[:, H:2 * H]
    yr0[...] = P[:, 2 * H:3 * H]
    yr1[...] = P[:, 3 * H:4 * H]
    zp[...] = P[:, 4 * H:4 * H + D]


def _dense(h_a, h_p, Wa, Wp):
    nh = jax.ShapeDtypeStruct((N, H), jnp.float32)
    nd = jax.ShapeDtypeStruct((N, D), jnp.float32)
    return pl.pallas_call(
        _dense_body,
        grid=(N // RB,),
        in_specs=[
            pl.BlockSpec((RB, D), lambda i: (i, 0)),
            pl.BlockSpec((RB, D), lambda i: (i, 0)),
            pl.BlockSpec((D, 2 * H + D), lambda i: (0, 0)),
            pl.BlockSpec((D, 4 * H + D), lambda i: (0, 0)),
        ],
        out_specs=[
            pl.BlockSpec((RB, H), lambda i: (i, 0)),
            pl.BlockSpec((RB, H), lambda i: (i, 0)),
            pl.BlockSpec((RB, D), lambda i: (i, 0)),
            pl.BlockSpec((RB, H), lambda i: (i, 0)),
            pl.BlockSpec((RB, H), lambda i: (i, 0)),
            pl.BlockSpec((RB, H), lambda i: (i, 0)),
            pl.BlockSpec((RB, H), lambda i: (i, 0)),
            pl.BlockSpec((RB, D), lambda i: (i, 0)),
        ],
        out_shape=[nh, nh, nd, nh, nh, nh, nh, nd],
    )(h_a, h_p, Wa, Wp)


def _seg_body(ta0, ta1, tc0, tc1, tr0, tr1,
              sw2, dw2, sc2, dc2, sr2, dr2, zeros_h,
              ow0, ow1, oc0, oc1, or0, or1,
              srcb, dstb, rows, acc, sem):
    cid = lax.axis_index("c")
    sid = lax.axis_index("s")

    def run(table, s2, d2, out_h):
        pltpu.sync_copy(zeros_h.at[pl.ds(sid * ZROWS, ZROWS)],
                        acc.at[pl.ds(sid * ZROWS, ZROWS)])
        plsc.subcore_barrier()

        def grp(g, carry):
            r0 = sid * ROWS_PER_TILE + g * KJ
            pltpu.sync_copy(s2.at[pl.ds(r0, KJ)], srcb)
            pltpu.sync_copy(d2.at[pl.ds(r0, KJ)], dstb)
            for j in range(KJ):
                pltpu.async_copy(table.at[srcb.at[j]], rows, sem).wait()
                pltpu.sync_copy(rows, acc.at[dstb.at[j]], add=True)
            return carry

        lax.fori_loop(0, NGROUP, grp, 0)
        plsc.subcore_barrier()
        pltpu.sync_copy(acc.at[pl.ds(sid * ZROWS, ZROWS)],
                        out_h.at[pl.ds(sid * ZROWS, ZROWS)])

    for c in range(2):
        @pl.when(cid == c)
        def _():
            tabs = (ta0, tc0, tr0) if c == 0 else (ta1, tc1, tr1)
            outs = (ow0, oc0, or0) if c == 0 else (ow1, oc1, or1)
            run(tabs[0], sw2, dw2, outs[0])
            run(tabs[1], sc2, dc2, outs[1])
            run(tabs[2], sr2, dr2, outs[2])


_segsum = functools.partial(
    pl.kernel,
    out_type=[jax.ShapeDtypeStruct((NACC, H), jnp.float32) for _ in range(6)],
    mesh=_MESH,
    scratch_types=[
        pltpu.VMEM((KJ, 128), jnp.int32),
        pltpu.VMEM((KJ, 128), jnp.int32),
        pltpu.VMEM((128, H), jnp.float32),
        pltpu.VMEM_SHARED((NACC, H), jnp.float32),
        pltpu.SemaphoreType.DMA,
    ],
)(_seg_body)


def _cnt_body(dw2, dc2, dr2, zeros_h, ones_h,
              cw, cc, cr, dstb, onesb, acc):
    cid = lax.axis_index("c")
    sid = lax.axis_index("s")
    pltpu.sync_copy(ones_h, onesb)

    def run(d2, out_h, flush_core):
        pltpu.sync_copy(zeros_h.at[pl.ds(sid * ZROWS, ZROWS)],
                        acc.at[pl.ds(sid * ZROWS, ZROWS)])
        plsc.subcore_barrier()

        def grp(g, carry):
            r0 = sid * ROWS_PER_TILE + g * KJ
            pltpu.sync_copy(d2.at[pl.ds(r0, KJ)], dstb)
            for j in range(KJ):
                pltpu.sync_copy(onesb, acc.at[dstb.at[j]], add=True)
            return carry

        lax.fori_loop(0, NGROUP, grp, 0)
        plsc.subcore_barrier()

        @pl.when(cid == flush_core)
        def _():
            pltpu.sync_copy(acc.at[pl.ds(sid * ZROWS, ZROWS)],
                            out_h.at[pl.ds(sid * ZROWS, ZROWS)])

    @pl.when(cid == 0)
    def _():
        run(dw2, cw, 0)

    @pl.when(cid == 1)
    def _():
        run(dc2, cc, 1)

    run(dr2, cr, 1)


_counts = functools.partial(
    pl.kernel,
    out_type=[jax.ShapeDtypeStruct((NACC, H), jnp.float32) for _ in range(3)],
    mesh=_MESH,
    scratch_types=[
        pltpu.VMEM((KJ, 128), jnp.int32),
        pltpu.VMEM((128, H), jnp.float32),
        pltpu.VMEM_SHARED((NACC, H), jnp.float32),
    ],
)(_cnt_body)


def _epi_body(sw0, sw1, sc0, sc1, sr0, sr1, zp, za, cw, cc, cr,
              bw, bc, br, hp_out, ha_out):
    Sw = jnp.concatenate([sw0[...], sw1[...]], axis=1)
    Sc = jnp.concatenate([sc0[...], sc1[...]], axis=1)
    Sr = jnp.concatenate([sr0[...], sr1[...]], axis=1)
    rw = 1.0 / jnp.maximum(cw[...][:, 0:1], 1.0)
    rc = 1.0 / jnp.maximum(cc[...][:, 0:1], 1.0)
    rr = 1.0 / jnp.maximum(cr[...][:, 0:1], 1.0)
    opp = (Sw * rw + bw[...] + Sc * rc + bc[...] + zp[...]) * 0.5
    hp_out[...] = ALPHA * jnp.where(opp >= 0, opp, NEG * opp)
    oa = Sr * rr + br[...] + za[...]
    ha_out[...] = ALPHA * jnp.where(oa >= 0, oa, NEG * oa)


def _epilogue(S, zp, za, cnts, bw, bc, br):
    blk_h = pl.BlockSpec((RB, H), lambda i: (i, 0))
    blk_d = pl.BlockSpec((RB, D), lambda i: (i, 0))
    blk_c = pl.BlockSpec((RB, H), lambda i: (i, 0))
    blk_b = pl.BlockSpec((1, D), lambda i: (0, 0))
    nd = jax.ShapeDtypeStruct((N, D), jnp.float32)
    return pl.pallas_call(
        _epi_body,
        grid=(N // RB,),
        in_specs=[blk_h] * 6 + [blk_d, blk_d] + [blk_c] * 3 + [blk_b] * 3,
        out_specs=[blk_d, blk_d],
        out_shape=[nd, nd],
    )(*S, zp, za, *cnts, bw, bc, br)


def _prep_edges(ei):
    pad = EPAD - E
    ar = jnp.arange(pad, dtype=jnp.int32)
    src = jnp.concatenate([ei[0].astype(jnp.int32), ar % 16])
    dst = jnp.concatenate([ei[1].astype(jnp.int32), N + (ar % NPADROW)])
    return src.reshape(EROWS, 128), dst.reshape(EROWS, 128)


def kernel(x_author, x_paper, ei_writes, ei_cites, ei_rev,
           Wl_0_writes, bl_0_writes, Wr_0_writes,
           Wl_0_cites, bl_0_cites, Wr_0_cites,
           Wl_0_rev, bl_0_rev, Wr_0_rev,
           Wl_1_writes, bl_1_writes, Wr_1_writes,
           Wl_1_cites, bl_1_cites, Wr_1_cites,
           Wl_1_rev, bl_1_rev, Wr_1_rev):
    sw2, dw2 = _prep_edges(ei_writes)
    sc2, dc2 = _prep_edges(ei_cites)
    sr2, dr2 = _prep_edges(ei_rev)
    zeros_h = jnp.zeros((NACC, H), jnp.float32)
    ones_h = jnp.ones((128, H), jnp.float32)

    cnts = _counts(dw2, dc2, dr2, zeros_h, ones_h)

    params = [
        (Wl_0_writes, bl_0_writes, Wr_0_writes, Wl_0_cites, bl_0_cites,
         Wr_0_cites, Wl_0_rev, bl_0_rev, Wr_0_rev),
        (Wl_1_writes, bl_1_writes, Wr_1_writes, Wl_1_cites, bl_1_cites,
         Wr_1_cites, Wl_1_rev, bl_1_rev, Wr_1_rev),
    ]
    h_a, h_p = x_author, x_paper
    for l in range(2):
        Wlw, bw, Wrw, Wlc, bc, Wrc, Wlr, br, Wrr = params[l]
        Wa = jnp.concatenate([Wlw.T, Wrr.T], axis=1)
        Wp = jnp.concatenate([Wlc.T, Wlr.T, (Wrw + Wrc).T], axis=1)
        if l == 0:
            Wa = Wa * ALPHA
            Wp = Wp * ALPHA
        yw0, yw1, za, yc0, yc1, yr0, yr1, zp = _dense(h_a, h_p, Wa, Wp)
        S = _segsum(yw0, yw1, yc0, yc1, yr0, yr1,
                    sw2, dw2, sc2, dc2, sr2, dr2, zeros_h)
        h_p, h_a = _epilogue(S, zp, za, cnts,
                             bw.reshape(1, D), bc.reshape(1, D),
                             br.reshape(1, D))
    return h_a, h_p


# speedup vs baseline: 5.8081x; 1.4745x over previous
import functools

import jax
import jax.numpy as jnp
from jax import lax
from jax.experimental import pallas as pl
from jax.experimental.pallas import tpu as pltpu
from jax.experimental.pallas import tpu_sc as plsc

N = 10000
D = 256
H = 128
E = 160000
EPAD = 163840
EROWS = EPAD // 128
ROWS_PER_TILE = EROWS // 16
KJ = 8
NGROUP = ROWS_PER_TILE // KJ
SP = 40
NPADROW = 16
NACC = 10112
ZROWS = NACC // 16
RB = 1000
ALPHA = 0.9999950000374997
NEG = 0.01

_MESH = plsc.VectorSubcoreMesh(core_axis_name="c", subcore_axis_name="s",
                               num_cores=2, num_subcores=16)


def _dense_body(a_ref, p_ref, wa_ref, wp_ref,
                yw0, yw1, za, yc0, yc1, yr0, yr1, zp):
    A = jnp.dot(a_ref[...], wa_ref[...], preferred_element_type=jnp.float32)
    P = jnp.dot(p_ref[...], wp_ref[...], preferred_element_type=jnp.float32)
    yw0[...] = A[:, 0:H]
    yw1[...] = A[:, H:2 * H]
    za[...] = A[:, 2 * H:2 * H + D]
    yc0[...] = P[:, 0:H]
    yc1[...] = P[:, H:2 * H]
    yr0[...] = P[:, 2 * H:3 * H]
    yr1[...] = P[:, 3 * H:4 * H]
    zp[...] = P[:, 4 * H:4 * H + D]


def _dense(h_a, h_p, Wa, Wp):
    nh = jax.ShapeDtypeStruct((N, H), jnp.float32)
    nd = jax.ShapeDtypeStruct((N, D), jnp.float32)
    return pl.pallas_call(
        _dense_body,
        grid=(N // RB,),
        in_specs=[
            pl.BlockSpec((RB, D), lambda i: (i, 0)),
            pl.BlockSpec((RB, D), lambda i: (i, 0)),
            pl.BlockSpec((D, 2 * H + D), lambda i: (0, 0)),
            pl.BlockSpec((D, 4 * H + D), lambda i: (0, 0)),
        ],
        out_specs=[
            pl.BlockSpec((RB, H), lambda i: (i, 0)),
            pl.BlockSpec((RB, H), lambda i: (i, 0)),
            pl.BlockSpec((RB, D), lambda i: (i, 0)),
            pl.BlockSpec((RB, H), lambda i: (i, 0)),
            pl.BlockSpec((RB, H), lambda i: (i, 0)),
            pl.BlockSpec((RB, H), lambda i: (i, 0)),
            pl.BlockSpec((RB, H), lambda i: (i, 0)),
            pl.BlockSpec((RB, D), lambda i: (i, 0)),
        ],
        out_shape=[nh, nh, nd, nh, nh, nh, nh, nd],
    )(h_a, h_p, Wa, Wp)


def _seg_body(ta0, ta1, tc0, tc1, tr0, tr1,
              sw2, dw2, sc2, dc2, sr2, dr2, zeros_h,
              ow0, ow1, oc0, oc1, or0, or1,
              srcb, dstb, rows0, rows1, acc, sem0, sem1):
    cid = lax.axis_index("c")
    sid = lax.axis_index("s")
    rows = (rows0, rows1)
    sems = (sem0, sem1)

    def run(table, s2, d2, out_h):
        pltpu.sync_copy(zeros_h.at[pl.ds(sid * ZROWS, ZROWS)],
                        acc.at[pl.ds(sid * ZROWS, ZROWS)])
        plsc.subcore_barrier()

        def fire(j, b):
            pltpu.async_copy(table.at[srcb.at[j]], rows[b], sems[b])

        def wait_scatter(j, b):
            pltpu.make_async_copy(table.at[srcb.at[j]], rows[b],
                                  sems[b]).wait()
            pltpu.sync_copy(rows[b], acc.at[dstb.at[j]], add=True)

        def body(k, carry):
            for b in range(2):
                j = 2 * k + b
                wait_scatter(j, b)
                fire(j + 2, b)
            return carry

        for half in range(ROWS_PER_TILE // SP):
            base = sid * ROWS_PER_TILE + half * SP
            pltpu.sync_copy(s2.at[pl.ds(base, SP)], srcb)
            pltpu.sync_copy(d2.at[pl.ds(base, SP)], dstb)
            fire(0, 0)
            fire(1, 1)
            lax.fori_loop(0, (SP - 2) // 2, body, 0)
            for b in range(2):
                wait_scatter(SP - 2 + b, b)

        plsc.subcore_barrier()
        pltpu.sync_copy(acc.at[pl.ds(sid * ZROWS, ZROWS)],
                        out_h.at[pl.ds(sid * ZROWS, ZROWS)])

    for c in range(2):
        @pl.when(cid == c)
        def _():
            tabs = (ta0, tc0, tr0) if c == 0 else (ta1, tc1, tr1)
            outs = (ow0, oc0, or0) if c == 0 else (ow1, oc1, or1)
            run(tabs[0], sw2, dw2, outs[0])
            run(tabs[1], sc2, dc2, outs[1])
            run(tabs[2], sr2, dr2, outs[2])


_segsum = functools.partial(
    pl.kernel,
    out_type=[jax.ShapeDtypeStruct((NACC, H), jnp.float32) for _ in range(6)],
    mesh=_MESH,
    scratch_types=[
        pltpu.VMEM((SP, 128), jnp.int32),
        pltpu.VMEM((SP, 128), jnp.int32),
        pltpu.VMEM((128, H), jnp.float32),
        pltpu.VMEM((128, H), jnp.float32),
        pltpu.VMEM_SHARED((NACC, H), jnp.float32),
        pltpu.SemaphoreType.DMA,
        pltpu.SemaphoreType.DMA,
    ],
)(_seg_body)


def _cnt_body(dw2, dc2, dr2, zeros_h, ones_h,
              cw, cc, cr, dstb, onesb, acc):
    cid = lax.axis_index("c")
    sid = lax.axis_index("s")
    pltpu.sync_copy(ones_h, onesb)

    def run(d2, out_h, flush_core):
        pltpu.sync_copy(zeros_h.at[pl.ds(sid * ZROWS, ZROWS)],
                        acc.at[pl.ds(sid * ZROWS, ZROWS)])
        plsc.subcore_barrier()

        def grp(g, carry):
            r0 = sid * ROWS_PER_TILE + g * KJ
            pltpu.sync_copy(d2.at[pl.ds(r0, KJ)], dstb)
            for j in range(KJ):
                pltpu.sync_copy(onesb, acc.at[dstb.at[j]], add=True)
            return carry

        lax.fori_loop(0, NGROUP, grp, 0)
        plsc.subcore_barrier()

        @pl.when(cid == flush_core)
        def _():
            pltpu.sync_copy(acc.at[pl.ds(sid * ZROWS, ZROWS)],
                            out_h.at[pl.ds(sid * ZROWS, ZROWS)])

    @pl.when(cid == 0)
    def _():
        run(dw2, cw, 0)

    @pl.when(cid == 1)
    def _():
        run(dc2, cc, 1)

    run(dr2, cr, 1)


_counts = functools.partial(
    pl.kernel,
    out_type=[jax.ShapeDtypeStruct((NACC, H), jnp.float32) for _ in range(3)],
    mesh=_MESH,
    scratch_types=[
        pltpu.VMEM((KJ, 128), jnp.int32),
        pltpu.VMEM((128, H), jnp.float32),
        pltpu.VMEM_SHARED((NACC, H), jnp.float32),
    ],
)(_cnt_body)


def _epi_body(sw0, sw1, sc0, sc1, sr0, sr1, zp, za, cw, cc, cr,
              bw, bc, br, hp_out, ha_out):
    Sw = jnp.concatenate([sw0[...], sw1[...]], axis=1)
    Sc = jnp.concatenate([sc0[...], sc1[...]], axis=1)
    Sr = jnp.concatenate([sr0[...], sr1[...]], axis=1)
    rw = 1.0 / jnp.maximum(cw[...][:, 0:1], 1.0)
    rc = 1.0 / jnp.maximum(cc[...][:, 0:1], 1.0)
    rr = 1.0 / jnp.maximum(cr[...][:, 0:1], 1.0)
    opp = (Sw * rw + bw[...] + Sc * rc + bc[...] + zp[...]) * 0.5
    hp_out[...] = ALPHA * jnp.where(opp >= 0, opp, NEG * opp)
    oa = Sr * rr + br[...] + za[...]
    ha_out[...] = ALPHA * jnp.where(oa >= 0, oa, NEG * oa)


def _epilogue(S, zp, za, cnts, bw, bc, br):
    blk_h = pl.BlockSpec((RB, H), lambda i: (i, 0))
    blk_d = pl.BlockSpec((RB, D), lambda i: (i, 0))
    blk_c = pl.BlockSpec((RB, H), lambda i: (i, 0))
    blk_b = pl.BlockSpec((1, D), lambda i: (0, 0))
    nd = jax.ShapeDtypeStruct((N, D), jnp.float32)
    return pl.pallas_call(
        _epi_body,
        grid=(N // RB,),
        in_specs=[blk_h] * 6 + [blk_d, blk_d] + [blk_c] * 3 + [blk_b] * 3,
        out_specs=[blk_d, blk_d],
        out_shape=[nd, nd],
    )(*S, zp, za, *cnts, bw, bc, br)


def _prep_edges(ei):
    pad = EPAD - E
    ar = jnp.arange(pad, dtype=jnp.int32)
    src = jnp.concatenate([ei[0].astype(jnp.int32), ar % 16])
    dst = jnp.concatenate([ei[1].astype(jnp.int32), N + (ar % NPADROW)])
    return src.reshape(EROWS, 128), dst.reshape(EROWS, 128)


def kernel(x_author, x_paper, ei_writes, ei_cites, ei_rev,
           Wl_0_writes, bl_0_writes, Wr_0_writes,
           Wl_0_cites, bl_0_cites, Wr_0_cites,
           Wl_0_rev, bl_0_rev, Wr_0_rev,
           Wl_1_writes, bl_1_writes, Wr_1_writes,
           Wl_1_cites, bl_1_cites, Wr_1_cites,
           Wl_1_rev, bl_1_rev, Wr_1_rev):
    sw2, dw2 = _prep_edges(ei_writes)
    sc2, dc2 = _prep_edges(ei_cites)
    sr2, dr2 = _prep_edges(ei_rev)
    zeros_h = jnp.zeros((NACC, H), jnp.float32)
    ones_h = jnp.ones((128, H), jnp.float32)

    cnts = _counts(dw2, dc2, dr2, zeros_h, ones_h)

    params = [
        (Wl_0_writes, bl_0_writes, Wr_0_writes, Wl_0_cites, bl_0_cites,
         Wr_0_cites, Wl_0_rev, bl_0_rev, Wr_0_rev),
        (Wl_1_writes, bl_1_writes, Wr_1_writes, Wl_1_cites, bl_1_cites,
         Wr_1_cites, Wl_1_rev, bl_1_rev, Wr_1_rev),
    ]
    h_a, h_p = x_author, x_paper
    for l in range(2):
        Wlw, bw, Wrw, Wlc, bc, Wrc, Wlr, br, Wrr = params[l]
        Wa = jnp.concatenate([Wlw.T, Wrr.T], axis=1)
        Wp = jnp.concatenate([Wlc.T, Wlr.T, (Wrw + Wrc).T], axis=1)
        if l == 0:
            Wa = Wa * ALPHA
            Wp = Wp * ALPHA
        yw0, yw1, za, yc0, yc1, yr0, yr1, zp = _dense(h_a, h_p, Wa, Wp)
        S = _segsum(yw0, yw1, yc0, yc1, yr0, yr1,
                    sw2, dw2, sc2, dc2, sr2, dr2, zeros_h)
        h_p, h_a = _epilogue(S, zp, za, cnts,
                             bw.reshape(1, D), bc.reshape(1, D),
                             br.reshape(1, D))
    return h_a, h_p


# speedup vs baseline: 6.1544x; 1.0596x over previous
import functools

import jax
import jax.numpy as jnp
from jax import lax
from jax.experimental import pallas as pl
from jax.experimental.pallas import tpu as pltpu
from jax.experimental.pallas import tpu_sc as plsc

N = 10000
D = 256
H = 128
E = 160000
EPAD = 163840
EROWS = EPAD // 128
ROWS_PER_TILE = EROWS // 16
KJ = 8
NGROUP = ROWS_PER_TILE // KJ
SP = 40
NPADROW = 16
NACC = 10112
ZROWS = NACC // 16
RB = 1000
ALPHA = 0.9999950000374997
NEG = 0.01

_MESH = plsc.VectorSubcoreMesh(core_axis_name="c", subcore_axis_name="s",
                               num_cores=2, num_subcores=16)


def _dense_body(a_ref, p_ref, wa_ref, wp_ref,
                yw0, yw1, za, yc0, yc1, yr0, yr1, zp):
    A = jnp.dot(a_ref[...], wa_ref[...], preferred_element_type=jnp.float32)
    P = jnp.dot(p_ref[...], wp_ref[...], preferred_element_type=jnp.float32)
    yw0[...] = A[:, 0:H]
    yw1[...] = A[:, H:2 * H]
    za[...] = A[:, 2 * H:2 * H + D]
    yc0[...] = P[:, 0:H]
    yc1[...] = P[:, H:2 * H]
    yr0[...] = P[:, 2 * H:3 * H]
    yr1[...] = P[:, 3 * H:4 * H]
    zp[...] = P[:, 4 * H:4 * H + D]


def _dense(h_a, h_p, Wa, Wp):
    nh = jax.ShapeDtypeStruct((N, H), jnp.float32)
    nd = jax.ShapeDtypeStruct((N, D), jnp.float32)
    return pl.pallas_call(
        _dense_body,
        grid=(N // RB,),
        in_specs=[
            pl.BlockSpec((RB, D), lambda i: (i, 0)),
            pl.BlockSpec((RB, D), lambda i: (i, 0)),
            pl.BlockSpec((D, 2 * H + D), lambda i: (0, 0)),
            pl.BlockSpec((D, 4 * H + D), lambda i: (0, 0)),
        ],
        out_specs=[
            pl.BlockSpec((RB, H), lambda i: (i, 0)),
            pl.BlockSpec((RB, H), lambda i: (i, 0)),
            pl.BlockSpec((RB, D), lambda i: (i, 0)),
            pl.BlockSpec((RB, H), lambda i: (i, 0)),
            pl.BlockSpec((RB, H), lambda i: (i, 0)),
            pl.BlockSpec((RB, H), lambda i: (i, 0)),
            pl.BlockSpec((RB, H), lambda i: (i, 0)),
            pl.BlockSpec((RB, D), lambda i: (i, 0)),
        ],
        out_shape=[nh, nh, nd, nh, nh, nh, nh, nd],
    )(h_a, h_p, Wa, Wp)


def _seg_body(ta0, ta1, tc0, tc1, tr0, tr1,
              sw2, dw2, sc2, dc2, sr2, dr2, zeros_h,
              ow0, ow1, oc0, oc1, or0, or1,
              srcb, dstb, rows0, rows1, acc, sem0, sem1):
    cid = lax.axis_index("c")
    sid = lax.axis_index("s")
    rows = (rows0, rows1)
    sems = (sem0, sem1)

    def run(table, s2, d2, out_h):
        pltpu.sync_copy(zeros_h.at[pl.ds(sid * ZROWS, ZROWS)],
                        acc.at[pl.ds(sid * ZROWS, ZROWS)])
        plsc.subcore_barrier()

        def fire(j, b):
            pltpu.async_copy(table.at[srcb.at[j]], rows[b], sems[b])

        def wait_scatter(j, b):
            pltpu.make_async_copy(table.at[srcb.at[j]], rows[b],
                                  sems[b]).wait()
            pltpu.sync_copy(rows[b], acc.at[dstb.at[j]], add=True)

        def body(k, carry):
            for b in range(2):
                j = 2 * k + b
                wait_scatter(j, b)
                fire(j + 2, b)
            return carry

        for half in range(ROWS_PER_TILE // SP):
            base = sid * ROWS_PER_TILE + half * SP
            pltpu.sync_copy(s2.at[pl.ds(base, SP)], srcb)
            pltpu.sync_copy(d2.at[pl.ds(base, SP)], dstb)
            fire(0, 0)
            fire(1, 1)
            lax.fori_loop(0, (SP - 2) // 2, body, 0)
            for b in range(2):
                wait_scatter(SP - 2 + b, b)

        plsc.subcore_barrier()
        pltpu.sync_copy(acc.at[pl.ds(sid * ZROWS, ZROWS)],
                        out_h.at[pl.ds(sid * ZROWS, ZROWS)])

    for c in range(2):
        @pl.when(cid == c)
        def _():
            tabs = (ta0, tc0, tr0) if c == 0 else (ta1, tc1, tr1)
            outs = (ow0, oc0, or0) if c == 0 else (ow1, oc1, or1)
            run(tabs[0], sw2, dw2, outs[0])
            run(tabs[1], sc2, dc2, outs[1])
            run(tabs[2], sr2, dr2, outs[2])


_segsum = functools.partial(
    pl.kernel,
    out_type=[jax.ShapeDtypeStruct((NACC, H), jnp.float32) for _ in range(6)],
    mesh=_MESH,
    scratch_types=[
        pltpu.VMEM((SP, 128), jnp.int32),
        pltpu.VMEM((SP, 128), jnp.int32),
        pltpu.VMEM((128, H), jnp.float32),
        pltpu.VMEM((128, H), jnp.float32),
        pltpu.VMEM_SHARED((NACC, H), jnp.float32),
        pltpu.SemaphoreType.DMA,
        pltpu.SemaphoreType.DMA,
    ],
)(_seg_body)


def _cnt_body(dw2, dc2, dr2, zeros_h, ones_h,
              cw, cc, cr0, cr1, dstb, onesb, acc):
    cid = lax.axis_index("c")
    sid = lax.axis_index("s")
    pltpu.sync_copy(ones_h, onesb)

    def run_half(d2, out0, out1):
        pltpu.sync_copy(zeros_h.at[pl.ds(sid * ZROWS, ZROWS)],
                        acc.at[pl.ds(sid * ZROWS, ZROWS)])
        plsc.subcore_barrier()

        def grp(g, carry):
            r0 = sid * ROWS_PER_TILE + cid * (ROWS_PER_TILE // 2) + g * KJ
            pltpu.sync_copy(d2.at[pl.ds(r0, KJ)], dstb)
            for j in range(KJ):
                pltpu.sync_copy(onesb, acc.at[dstb.at[j]], add=True)
            return carry

        lax.fori_loop(0, NGROUP // 2, grp, 0)
        plsc.subcore_barrier()

        @pl.when(cid == 0)
        def _():
            pltpu.sync_copy(acc.at[pl.ds(sid * ZROWS, ZROWS)],
                            out0.at[pl.ds(sid * ZROWS, ZROWS)])

        @pl.when(cid == 1)
        def _():
            pltpu.sync_copy(acc.at[pl.ds(sid * ZROWS, ZROWS)],
                            out1.at[pl.ds(sid * ZROWS, ZROWS)])

    def run(d2, out_h, flush_core):
        pltpu.sync_copy(zeros_h.at[pl.ds(sid * ZROWS, ZROWS)],
                        acc.at[pl.ds(sid * ZROWS, ZROWS)])
        plsc.subcore_barrier()

        def grp(g, carry):
            r0 = sid * ROWS_PER_TILE + g * KJ
            pltpu.sync_copy(d2.at[pl.ds(r0, KJ)], dstb)
            for j in range(KJ):
                pltpu.sync_copy(onesb, acc.at[dstb.at[j]], add=True)
            return carry

        lax.fori_loop(0, NGROUP, grp, 0)
        plsc.subcore_barrier()

        @pl.when(cid == flush_core)
        def _():
            pltpu.sync_copy(acc.at[pl.ds(sid * ZROWS, ZROWS)],
                            out_h.at[pl.ds(sid * ZROWS, ZROWS)])

    @pl.when(cid == 0)
    def _():
        run(dw2, cw, 0)

    @pl.when(cid == 1)
    def _():
        run(dc2, cc, 1)

    run_half(dr2, cr0, cr1)


_counts = functools.partial(
    pl.kernel,
    out_type=[jax.ShapeDtypeStruct((NACC, H), jnp.float32) for _ in range(4)],
    mesh=_MESH,
    scratch_types=[
        pltpu.VMEM((KJ, 128), jnp.int32),
        pltpu.VMEM((128, H), jnp.float32),
        pltpu.VMEM_SHARED((NACC, H), jnp.float32),
    ],
)(_cnt_body)


def _combine(sw0, sw1, sc0, sc1, sr0, sr1, zp, za, cw, cc, cr0, cr1,
             bw, bc, br):
    Sw = jnp.concatenate([sw0[...], sw1[...]], axis=1)
    Sc = jnp.concatenate([sc0[...], sc1[...]], axis=1)
    Sr = jnp.concatenate([sr0[...], sr1[...]], axis=1)
    rw = 1.0 / jnp.maximum(cw[...][:, 0:1], 1.0)
    rc = 1.0 / jnp.maximum(cc[...][:, 0:1], 1.0)
    rr = 1.0 / jnp.maximum(cr0[...][:, 0:1] + cr1[...][:, 0:1], 1.0)
    opp = (Sw * rw + bw[...] + Sc * rc + bc[...] + zp[...]) * 0.5
    hp = ALPHA * jnp.where(opp >= 0, opp, NEG * opp)
    oa = Sr * rr + br[...] + za[...]
    ha = ALPHA * jnp.where(oa >= 0, oa, NEG * oa)
    return hp, ha


def _epi_body(sw0, sw1, sc0, sc1, sr0, sr1, zp, za, cw, cc, cr0, cr1,
              bw, bc, br, hp_out, ha_out):
    hp, ha = _combine(sw0, sw1, sc0, sc1, sr0, sr1, zp, za,
                      cw, cc, cr0, cr1, bw, bc, br)
    hp_out[...] = hp
    ha_out[...] = ha


def _epi_dense_body(sw0, sw1, sc0, sc1, sr0, sr1, zp, za, cw, cc, cr0, cr1,
                    bw, bc, br, wa_ref, wp_ref,
                    yw0, yw1, za_o, yc0, yc1, yr0, yr1, zp_o):
    hp, ha = _combine(sw0, sw1, sc0, sc1, sr0, sr1, zp, za,
                      cw, cc, cr0, cr1, bw, bc, br)
    A = jnp.dot(ha, wa_ref[...], preferred_element_type=jnp.float32)
    P = jnp.dot(hp, wp_ref[...], preferred_element_type=jnp.float32)
    yw0[...] = A[:, 0:H]
    yw1[...] = A[:, H:2 * H]
    za_o[...] = A[:, 2 * H:2 * H + D]
    yc0[...] = P[:, 0:H]
    yc1[...] = P[:, H:2 * H]
    yr0[...] = P[:, 2 * H:3 * H]
    yr1[...] = P[:, 3 * H:4 * H]
    zp_o[...] = P[:, 4 * H:4 * H + D]


def _epi_in_specs():
    blk_h = pl.BlockSpec((RB, H), lambda i: (i, 0))
    blk_d = pl.BlockSpec((RB, D), lambda i: (i, 0))
    blk_b = pl.BlockSpec((1, D), lambda i: (0, 0))
    return [blk_h] * 6 + [blk_d, blk_d] + [blk_h] * 4 + [blk_b] * 3


def _epilogue(S, zp, za, cnts, bw, bc, br):
    blk_d = pl.BlockSpec((RB, D), lambda i: (i, 0))
    nd = jax.ShapeDtypeStruct((N, D), jnp.float32)
    return pl.pallas_call(
        _epi_body,
        grid=(N // RB,),
        in_specs=_epi_in_specs(),
        out_specs=[blk_d, blk_d],
        out_shape=[nd, nd],
    )(*S, zp, za, *cnts, bw, bc, br)


def _epi_dense(S, zp, za, cnts, bw, bc, br, Wa, Wp):
    blk_h = pl.BlockSpec((RB, H), lambda i: (i, 0))
    blk_d = pl.BlockSpec((RB, D), lambda i: (i, 0))
    nh = jax.ShapeDtypeStruct((N, H), jnp.float32)
    nd = jax.ShapeDtypeStruct((N, D), jnp.float32)
    return pl.pallas_call(
        _epi_dense_body,
        grid=(N // RB,),
        in_specs=_epi_in_specs() + [
            pl.BlockSpec((D, 2 * H + D), lambda i: (0, 0)),
            pl.BlockSpec((D, 4 * H + D), lambda i: (0, 0)),
        ],
        out_specs=[blk_h, blk_h, blk_d, blk_h, blk_h, blk_h, blk_h, blk_d],
        out_shape=[nh, nh, nd, nh, nh, nh, nh, nd],
    )(*S, zp, za, *cnts, bw, bc, br, Wa, Wp)


def _prep_edges(ei):
    pad = EPAD - E
    ar = jnp.arange(pad, dtype=jnp.int32)
    src = jnp.concatenate([ei[0].astype(jnp.int32), ar % 16])
    dst = jnp.concatenate([ei[1].astype(jnp.int32), N + (ar % NPADROW)])
    return src.reshape(EROWS, 128), dst.reshape(EROWS, 128)


def kernel(x_author, x_paper, ei_writes, ei_cites, ei_rev,
           Wl_0_writes, bl_0_writes, Wr_0_writes,
           Wl_0_cites, bl_0_cites, Wr_0_cites,
           Wl_0_rev, bl_0_rev, Wr_0_rev,
           Wl_1_writes, bl_1_writes, Wr_1_writes,
           Wl_1_cites, bl_1_cites, Wr_1_cites,
           Wl_1_rev, bl_1_rev, Wr_1_rev):
    sw2, dw2 = _prep_edges(ei_writes)
    sc2, dc2 = _prep_edges(ei_cites)
    sr2, dr2 = _prep_edges(ei_rev)
    zeros_h = jnp.zeros((NACC, H), jnp.float32)
    ones_h = jnp.ones((128, H), jnp.float32)

    cnts = _counts(dw2, dc2, dr2, zeros_h, ones_h)

    Wa0 = jnp.concatenate([Wl_0_writes.T, Wr_0_rev.T], axis=1) * ALPHA
    Wp0 = jnp.concatenate(
        [Wl_0_cites.T, Wl_0_rev.T, (Wr_0_writes + Wr_0_cites).T],
        axis=1) * ALPHA
    Wa1 = jnp.concatenate([Wl_1_writes.T, Wr_1_rev.T], axis=1)
    Wp1 = jnp.concatenate(
        [Wl_1_cites.T, Wl_1_rev.T, (Wr_1_writes + Wr_1_cites).T], axis=1)

    yw0, yw1, za, yc0, yc1, yr0, yr1, zp = _dense(x_author, x_paper, Wa0, Wp0)
    S0 = _segsum(yw0, yw1, yc0, yc1, yr0, yr1,
                 sw2, dw2, sc2, dc2, sr2, dr2, zeros_h)
    yw0, yw1, za1, yc0, yc1, yr0, yr1, zp1 = _epi_dense(
        S0, zp, za, cnts, bl_0_writes.reshape(1, D),
        bl_0_cites.reshape(1, D), bl_0_rev.reshape(1, D), Wa1, Wp1)
    S1 = _segsum(yw0, yw1, yc0, yc1, yr0, yr1,
                 sw2, dw2, sc2, dc2, sr2, dr2, zeros_h)
    h_p, h_a = _epilogue(S1, zp1, za1, cnts,
                         bl_1_writes.reshape(1, D),
                         bl_1_cites.reshape(1, D),
                         bl_1_rev.reshape(1, D))
    return h_a, h_p


# speedup vs baseline: 6.2063x; 1.0084x over previous
import functools

import jax
import jax.numpy as jnp
from jax import lax
from jax.experimental import pallas as pl
from jax.experimental.pallas import tpu as pltpu
from jax.experimental.pallas import tpu_sc as plsc

N = 10000
D = 256
H = 128
E = 160000
EPAD = 163840
EROWS = EPAD // 128
ROWS_PER_TILE = EROWS // 16
KJ = 8
NGROUP = ROWS_PER_TILE // KJ
SP = 40
NPADROW = 16
NACC = 10112
ZROWS = NACC // 16
RB = 1000
ALPHA = 0.9999950000374997
NEG = 0.01

_MESH = plsc.VectorSubcoreMesh(core_axis_name="c", subcore_axis_name="s",
                               num_cores=2, num_subcores=16)


def _dense_body(a_ref, p_ref, wa_ref, wp_ref,
                yw0, yw1, za, yc0, yc1, yr0, yr1, zp):
    A = jnp.dot(a_ref[...], wa_ref[...], preferred_element_type=jnp.float32)
    P = jnp.dot(p_ref[...], wp_ref[...], preferred_element_type=jnp.float32)
    yw0[...] = A[:, 0:H]
    yw1[...] = A[:, H:2 * H]
    za[...] = A[:, 2 * H:2 * H + D]
    yc0[...] = P[:, 0:H]
    yc1[...] = P[:, H:2 * H]
    yr0[...] = P[:, 2 * H:3 * H]
    yr1[...] = P[:, 3 * H:4 * H]
    zp[...] = P[:, 4 * H:4 * H + D]


def _dense(h_a, h_p, Wa, Wp):
    nh = jax.ShapeDtypeStruct((N, H), jnp.float32)
    nd = jax.ShapeDtypeStruct((N, D), jnp.float32)
    return pl.pallas_call(
        _dense_body,
        grid=(N // RB,),
        in_specs=[
            pl.BlockSpec((RB, D), lambda i: (i, 0)),
            pl.BlockSpec((RB, D), lambda i: (i, 0)),
            pl.BlockSpec((D, 2 * H + D), lambda i: (0, 0)),
            pl.BlockSpec((D, 4 * H + D), lambda i: (0, 0)),
        ],
        out_specs=[
            pl.BlockSpec((RB, H), lambda i: (i, 0)),
            pl.BlockSpec((RB, H), lambda i: (i, 0)),
            pl.BlockSpec((RB, D), lambda i: (i, 0)),
            pl.BlockSpec((RB, H), lambda i: (i, 0)),
            pl.BlockSpec((RB, H), lambda i: (i, 0)),
            pl.BlockSpec((RB, H), lambda i: (i, 0)),
            pl.BlockSpec((RB, H), lambda i: (i, 0)),
            pl.BlockSpec((RB, D), lambda i: (i, 0)),
        ],
        out_shape=[nh, nh, nd, nh, nh, nh, nh, nd],
    )(h_a, h_p, Wa, Wp)


def _seg_body(ta0, ta1, tc0, tc1, tr0, tr1,
              sw2, dw2, sc2, dc2, sr2, dr2, zeros_h,
              ow0, ow1, oc0, oc1, or0, or1,
              srcb, dstb, rows0, rows1, acc, sem0, sem1):
    cid = lax.axis_index("c")
    sid = lax.axis_index("s")
    rows = (rows0, rows1)
    sems = (sem0, sem1)

    def run(table, s2, d2, out_h):
        pltpu.sync_copy(zeros_h.at[pl.ds(sid * ZROWS, ZROWS)],
                        acc.at[pl.ds(sid * ZROWS, ZROWS)])
        plsc.subcore_barrier()

        def fire(j, b):
            pltpu.async_copy(table.at[srcb.at[j]], rows[b], sems[b])

        def wait_scatter(j, b):
            pltpu.make_async_copy(table.at[srcb.at[j]], rows[b],
                                  sems[b]).wait()
            pltpu.sync_copy(rows[b], acc.at[dstb.at[j]], add=True)

        def body(k, carry):
            for b in range(2):
                j = 2 * k + b
                wait_scatter(j, b)
                fire(j + 2, b)
            return carry

        for half in range(ROWS_PER_TILE // SP):
            base = sid * ROWS_PER_TILE + half * SP
            pltpu.sync_copy(s2.at[pl.ds(base, SP)], srcb)
            pltpu.sync_copy(d2.at[pl.ds(base, SP)], dstb)
            fire(0, 0)
            fire(1, 1)
            lax.fori_loop(0, (SP - 2) // 2, body, 0)
            for b in range(2):
                wait_scatter(SP - 2 + b, b)

        plsc.subcore_barrier()
        pltpu.sync_copy(acc.at[pl.ds(sid * ZROWS, ZROWS)],
                        out_h.at[pl.ds(sid * ZROWS, ZROWS)])

    for c in range(2):
        @pl.when(cid == c)
        def _():
            tabs = (ta0, tc0, tr0) if c == 0 else (ta1, tc1, tr1)
            outs = (ow0, oc0, or0) if c == 0 else (ow1, oc1, or1)
            run(tabs[0], sw2, dw2, outs[0])
            run(tabs[1], sc2, dc2, outs[1])
            run(tabs[2], sr2, dr2, outs[2])


_segsum = functools.partial(
    pl.kernel,
    out_type=[jax.ShapeDtypeStruct((NACC, H), jnp.float32) for _ in range(6)],
    mesh=_MESH,
    scratch_types=[
        pltpu.VMEM((SP, 128), jnp.int32),
        pltpu.VMEM((SP, 128), jnp.int32),
        pltpu.VMEM((128, H), jnp.float32),
        pltpu.VMEM((128, H), jnp.float32),
        pltpu.VMEM_SHARED((NACC, H), jnp.float32),
        pltpu.SemaphoreType.DMA,
        pltpu.SemaphoreType.DMA,
    ],
)(_seg_body)


def _cnt_body(dw2, dc2, dr2, zeros_h, ones_h,
              cw, cc, cr0, cr1, dstb, onesb, acc):
    cid = lax.axis_index("c")
    sid = lax.axis_index("s")
    pltpu.sync_copy(ones_h, onesb)

    def scat(j, carry):
        pltpu.sync_copy(onesb, acc.at[dstb.at[j]], add=True)
        return carry

    def run_half(d2, out0, out1):
        pltpu.sync_copy(zeros_h.at[pl.ds(sid * ZROWS, ZROWS)],
                        acc.at[pl.ds(sid * ZROWS, ZROWS)])
        r0 = sid * ROWS_PER_TILE + cid * SP
        pltpu.sync_copy(d2.at[pl.ds(r0, SP)], dstb)
        plsc.subcore_barrier()
        lax.fori_loop(0, SP, scat, 0)
        plsc.subcore_barrier()

        @pl.when(cid == 0)
        def _():
            pltpu.sync_copy(acc.at[pl.ds(sid * ZROWS, ZROWS)],
                            out0.at[pl.ds(sid * ZROWS, ZROWS)])

        @pl.when(cid == 1)
        def _():
            pltpu.sync_copy(acc.at[pl.ds(sid * ZROWS, ZROWS)],
                            out1.at[pl.ds(sid * ZROWS, ZROWS)])

    def run(d2, out_h, flush_core):
        pltpu.sync_copy(zeros_h.at[pl.ds(sid * ZROWS, ZROWS)],
                        acc.at[pl.ds(sid * ZROWS, ZROWS)])
        plsc.subcore_barrier()
        for half in range(ROWS_PER_TILE // SP):
            r0 = sid * ROWS_PER_TILE + half * SP
            pltpu.sync_copy(d2.at[pl.ds(r0, SP)], dstb)
            lax.fori_loop(0, SP, scat, 0)
        plsc.subcore_barrier()

        @pl.when(cid == flush_core)
        def _():
            pltpu.sync_copy(acc.at[pl.ds(sid * ZROWS, ZROWS)],
                            out_h.at[pl.ds(sid * ZROWS, ZROWS)])

    @pl.when(cid == 0)
    def _():
        run(dw2, cw, 0)

    @pl.when(cid == 1)
    def _():
        run(dc2, cc, 1)

    run_half(dr2, cr0, cr1)


_counts = functools.partial(
    pl.kernel,
    out_type=[jax.ShapeDtypeStruct((NACC, H), jnp.float32) for _ in range(4)],
    mesh=_MESH,
    scratch_types=[
        pltpu.VMEM((SP, 128), jnp.int32),
        pltpu.VMEM((128, H), jnp.float32),
        pltpu.VMEM_SHARED((NACC, H), jnp.float32),
    ],
)(_cnt_body)


def _combine(sw0, sw1, sc0, sc1, sr0, sr1, zp, za, cw, cc, cr0, cr1,
             bw, bc, br):
    Sw = jnp.concatenate([sw0[...], sw1[...]], axis=1)
    Sc = jnp.concatenate([sc0[...], sc1[...]], axis=1)
    Sr = jnp.concatenate([sr0[...], sr1[...]], axis=1)
    rw = 1.0 / jnp.maximum(cw[...][:, 0:1], 1.0)
    rc = 1.0 / jnp.maximum(cc[...][:, 0:1], 1.0)
    rr = 1.0 / jnp.maximum(cr0[...][:, 0:1] + cr1[...][:, 0:1], 1.0)
    opp = (Sw * rw + bw[...] + Sc * rc + bc[...] + zp[...]) * 0.5
    hp = ALPHA * jnp.where(opp >= 0, opp, NEG * opp)
    oa = Sr * rr + br[...] + za[...]
    ha = ALPHA * jnp.where(oa >= 0, oa, NEG * oa)
    return hp, ha


def _epi_body(sw0, sw1, sc0, sc1, sr0, sr1, zp, za, cw, cc, cr0, cr1,
              bw, bc, br, hp_out, ha_out):
    hp, ha = _combine(sw0, sw1, sc0, sc1, sr0, sr1, zp, za,
                      cw, cc, cr0, cr1, bw, bc, br)
    hp_out[...] = hp
    ha_out[...] = ha


def _epi_dense_body(sw0, sw1, sc0, sc1, sr0, sr1, zp, za, cw, cc, cr0, cr1,
                    bw, bc, br, wa_ref, wp_ref,
                    yw0, yw1, za_o, yc0, yc1, yr0, yr1, zp_o):
    hp, ha = _combine(sw0, sw1, sc0, sc1, sr0, sr1, zp, za,
                      cw, cc, cr0, cr1, bw, bc, br)
    A = jnp.dot(ha, wa_ref[...], preferred_element_type=jnp.float32)
    P = jnp.dot(hp, wp_ref[...], preferred_element_type=jnp.float32)
    yw0[...] = A[:, 0:H]
    yw1[...] = A[:, H:2 * H]
    za_o[...] = A[:, 2 * H:2 * H + D]
    yc0[...] = P[:, 0:H]
    yc1[...] = P[:, H:2 * H]
    yr0[...] = P[:, 2 * H:3 * H]
    yr1[...] = P[:, 3 * H:4 * H]
    zp_o[...] = P[:, 4 * H:4 * H + D]


def _epi_in_specs():
    blk_h = pl.BlockSpec((RB, H), lambda i: (i, 0))
    blk_d = pl.BlockSpec((RB, D), lambda i: (i, 0))
    blk_b = pl.BlockSpec((1, D), lambda i: (0, 0))
    return [blk_h] * 6 + [blk_d, blk_d] + [blk_h] * 4 + [blk_b] * 3


def _epilogue(S, zp, za, cnts, bw, bc, br):
    blk_d = pl.BlockSpec((RB, D), lambda i: (i, 0))
    nd = jax.ShapeDtypeStruct((N, D), jnp.float32)
    return pl.pallas_call(
        _epi_body,
        grid=(N // RB,),
        in_specs=_epi_in_specs(),
        out_specs=[blk_d, blk_d],
        out_shape=[nd, nd],
    )(*S, zp, za, *cnts, bw, bc, br)


def _epi_dense(S, zp, za, cnts, bw, bc, br, Wa, Wp):
    blk_h = pl.BlockSpec((RB, H), lambda i: (i, 0))
    blk_d = pl.BlockSpec((RB, D), lambda i: (i, 0))
    nh = jax.ShapeDtypeStruct((N, H), jnp.float32)
    nd = jax.ShapeDtypeStruct((N, D), jnp.float32)
    return pl.pallas_call(
        _epi_dense_body,
        grid=(N // RB,),
        in_specs=_epi_in_specs() + [
            pl.BlockSpec((D, 2 * H + D), lambda i: (0, 0)),
            pl.BlockSpec((D, 4 * H + D), lambda i: (0, 0)),
        ],
        out_specs=[blk_h, blk_h, blk_d, blk_h, blk_h, blk_h, blk_h, blk_d],
        out_shape=[nh, nh, nd, nh, nh, nh, nh, nd],
    )(*S, zp, za, *cnts, bw, bc, br, Wa, Wp)


def _prep_edges(ei):
    pad = EPAD - E
    ar = jnp.arange(pad, dtype=jnp.int32)
    src = jnp.concatenate([ei[0].astype(jnp.int32), ar % 16])
    dst = jnp.concatenate([ei[1].astype(jnp.int32), N + (ar % NPADROW)])
    return src.reshape(EROWS, 128), dst.reshape(EROWS, 128)


def kernel(x_author, x_paper, ei_writes, ei_cites, ei_rev,
           Wl_0_writes, bl_0_writes, Wr_0_writes,
           Wl_0_cites, bl_0_cites, Wr_0_cites,
           Wl_0_rev, bl_0_rev, Wr_0_rev,
           Wl_1_writes, bl_1_writes, Wr_1_writes,
           Wl_1_cites, bl_1_cites, Wr_1_cites,
           Wl_1_rev, bl_1_rev, Wr_1_rev):
    sw2, dw2 = _prep_edges(ei_writes)
    sc2, dc2 = _prep_edges(ei_cites)
    sr2, dr2 = _prep_edges(ei_rev)
    zeros_h = jnp.zeros((NACC, H), jnp.float32)
    ones_h = jnp.ones((128, H), jnp.float32)

    cnts = _counts(dw2, dc2, dr2, zeros_h, ones_h)

    Wa0 = jnp.concatenate([Wl_0_writes.T, Wr_0_rev.T], axis=1) * ALPHA
    Wp0 = jnp.concatenate(
        [Wl_0_cites.T, Wl_0_rev.T, (Wr_0_writes + Wr_0_cites).T],
        axis=1) * ALPHA
    Wa1 = jnp.concatenate([Wl_1_writes.T, Wr_1_rev.T], axis=1)
    Wp1 = jnp.concatenate(
        [Wl_1_cites.T, Wl_1_rev.T, (Wr_1_writes + Wr_1_cites).T], axis=1)

    yw0, yw1, za, yc0, yc1, yr0, yr1, zp = _dense(x_author, x_paper, Wa0, Wp0)
    S0 = _segsum(yw0, yw1, yc0, yc1, yr0, yr1,
                 sw2, dw2, sc2, dc2, sr2, dr2, zeros_h)
    yw0, yw1, za1, yc0, yc1, yr0, yr1, zp1 = _epi_dense(
        S0, zp, za, cnts, bl_0_writes.reshape(1, D),
        bl_0_cites.reshape(1, D), bl_0_rev.reshape(1, D), Wa1, Wp1)
    S1 = _segsum(yw0, yw1, yc0, yc1, yr0, yr1,
                 sw2, dw2, sc2, dc2, sr2, dr2, zeros_h)
    h_p, h_a = _epilogue(S1, zp1, za1, cnts,
                         bl_1_writes.reshape(1, D),
                         bl_1_cites.reshape(1, D),
                         bl_1_rev.reshape(1, D))
    return h_a, h_p
